# baseline (speedup 1.0000x reference)
import functools

import jax
import jax.numpy as jnp
from jax import lax
from jax.experimental import pallas as pl
from jax.experimental.pallas import tpu as pltpu
from jax.experimental.pallas import tpu_sc as plsc

N = 8192
M = 8192
BIG = 3.0e38
FAR = 1.0e18

NTC = 4608
NSC = N - NTC

NW = 32
BPW = NSC // NW
L = 16
RV = M // L
UNROLL = 8
GB = 4

BN = 512
BM = 2048


_GATHER_DNUMS = lax.GatherDimensionNumbers(
    offset_dims=(), collapsed_slice_dims=(0,), start_index_map=(0,))


def _shuffle(v, idx):
    return lax.gather(v, idx[:, None], _GATHER_DNUMS, slice_sizes=(1,),
                      mode=lax.GatherScatterMode.PROMISE_IN_BOUNDS)


def _lane_min_all(v):
    lane_iota = lax.iota(jnp.int32, L)
    for s in (8, 4, 2, 1):
        v = jnp.minimum(v, _shuffle(v, lane_iota ^ s))
    return v


def _sc_main(bx_hbm, by_hbm, rx_hbm, ry_hbm,
             rowmin_hbm, maskf_hbm, colpart_hbm,
             rx_v, ry_v, rn2_v, colacc_v, bx_v, by_v, rmin_v, mask_v):
    wid = lax.axis_index("c") * 16 + lax.axis_index("s")
    base = wid * BPW

    pltpu.sync_copy(rx_hbm, rx_v)
    pltpu.sync_copy(ry_hbm, ry_v)
    pltpu.sync_copy(bx_hbm.at[pl.ds(base, BPW)], bx_v)
    pltpu.sync_copy(by_hbm.at[pl.ds(base, BPW)], by_v)

    def mask_body(t, carry):
        sl = pl.ds(t * L, L)
        bxv = bx_v[sl]
        byv = by_v[sl]
        ok = ((bxv >= -2000.0) & (bxv <= 2000.0) &
              (byv >= -2000.0) & (byv <= 2000.0))
        mask_v[sl] = jnp.where(ok, jnp.float32(1.0), jnp.float32(0.0))
        bx_v[sl] = jnp.where(ok, bxv, jnp.float32(FAR))
        by_v[sl] = jnp.where(ok, byv, jnp.float32(FAR))
        return carry

    lax.fori_loop(0, BPW // L, mask_body, 0)

    def prep_body(j, carry):
        sl = pl.ds(j * L, L)
        rxv = rx_v[sl]
        ryv = ry_v[sl]
        rn2_v[sl] = rxv * rxv + ryv * ryv
        colacc_v[sl] = jnp.full((L,), BIG, jnp.float32)
        return carry

    lax.fori_loop(0, RV, prep_body, 0)

    lane_iota = lax.iota(jnp.int32, L)

    def super_body(t, carry):
        i0 = t * L
        bxv16 = bx_v[pl.ds(i0, L)]
        byv16 = by_v[pl.ds(i0, L)]
        rminvec = jnp.full((L,), BIG, jnp.float32)
        for sub in range(L // GB):
            nbx = []
            nby = []
            bn2 = []
            for k in range(GB):
                lane = sub * GB + k
                bxb = jnp.full((L,), bxv16[lane], jnp.float32)
                byb = jnp.full((L,), byv16[lane], jnp.float32)
                nbx.append(bxb * (-2.0))
                nby.append(byb * (-2.0))
                bn2.append(bxb * bxb + byb * byb)

            def inner(j, accs):
                accs = list(accs)
                sl = pl.ds(j * L, L)
                rxv = rx_v[sl]
                ryv = ry_v[sl]
                rn2v = rn2_v[sl]
                d2s = []
                for k in range(GB):
                    t1 = nbx[k] * rxv + rn2v
                    t2 = nby[k] * ryv + t1
                    accs[k] = jnp.minimum(accs[k], t2)
                    d2s.append(t2 + bn2[k])
                cm = jnp.minimum(jnp.minimum(d2s[0], d2s[1]),
                                 jnp.minimum(d2s[2], d2s[3]))
                colacc_v[sl] = jnp.minimum(colacc_v[sl], cm)
                return tuple(accs)

            accs = plsc.parallel_loop(
                0, RV, 1, unroll=UNROLL,
                carry=tuple(jnp.full((L,), BIG, jnp.float32)
                            for _ in range(GB)))(inner)
            for k in range(GB):
                lane = sub * GB + k
                red = _lane_min_all(accs[k]) + bn2[k]
                rminvec = jnp.where(lane_iota == lane, red, rminvec)
        rmin_v[pl.ds(i0, L)] = rminvec
        return carry

    lax.fori_loop(0, BPW // L, super_body, 0)

    pltpu.sync_copy(rmin_v, rowmin_hbm.at[pl.ds(base, BPW)])
    pltpu.sync_copy(mask_v, maskf_hbm.at[pl.ds(base, BPW)])
    pltpu.sync_copy(colacc_v, colpart_hbm.at[wid])


def _make_sc_call():
    return functools.partial(
        pl.kernel,
        out_type=(jax.ShapeDtypeStruct((NSC,), jnp.float32),
                  jax.ShapeDtypeStruct((NSC,), jnp.float32),
                  jax.ShapeDtypeStruct((NW, M), jnp.float32)),
        mesh=plsc.VectorSubcoreMesh(core_axis_name="c", subcore_axis_name="s",
                                    num_cores=2, num_subcores=16),
        scratch_types=[
            pltpu.VMEM((M,), jnp.float32),
            pltpu.VMEM((M,), jnp.float32),
            pltpu.VMEM((M,), jnp.float32),
            pltpu.VMEM((M,), jnp.float32),
            pltpu.VMEM((BPW,), jnp.float32),
            pltpu.VMEM((BPW,), jnp.float32),
            pltpu.VMEM((BPW,), jnp.float32),
            pltpu.VMEM((BPW,), jnp.float32),
        ],
    )(_sc_main)


def _tc_body(bx_ref, by_ref, rx_ref, ry_ref,
             rowmin_ref, maskf_ref, colpart_ref, rowacc, colacc,
             bxm_s, bym_s):
    i = pl.program_id(0)
    j = pl.program_id(1)
    ni = pl.num_programs(0)
    nj = pl.num_programs(1)

    rx = rx_ref[...]
    ry = ry_ref[...]

    @pl.when(j == 0)
    def _():
        bx = bx_ref[...]
        by = by_ref[...]
        mask = ((bx >= -2000.0) & (bx <= 2000.0) &
                (by >= -2000.0) & (by <= 2000.0))
        bxm_s[...] = jnp.where(mask, bx, FAR)
        bym_s[...] = jnp.where(mask, by, FAR)
        maskf_ref[...] = mask.astype(jnp.float32)

    bxm = bxm_s[...]
    bym = bym_s[...]

    dx = bxm - rx
    dy = bym - ry
    d2 = dx * dx + dy * dy

    rmin = jnp.min(d2, axis=1, keepdims=True)

    @pl.when(j == 0)
    def _():
        rowacc[...] = rmin

    @pl.when(j != 0)
    def _():
        rowacc[...] = jnp.minimum(rowacc[...], rmin)

    cmin = jnp.min(d2, axis=0, keepdims=True)

    @pl.when(i == 0)
    def _():
        colacc[0:1, pl.ds(j * BM, BM)] = cmin

    @pl.when(i != 0)
    def _():
        colacc[0:1, pl.ds(j * BM, BM)] = jnp.minimum(
            colacc[0:1, pl.ds(j * BM, BM)], cmin)

    @pl.when(j == nj - 1)
    def _():
        rowmin_ref[...] = rowacc[...]

    @pl.when((i == ni - 1) & (j == nj - 1))
    def _():
        colpart_ref[...] = colacc[...]


def _tc_call(bx, by, rx, ry):
    grid = (NTC // BN, M // BM)
    return pl.pallas_call(
        _tc_body,
        grid=grid,
        in_specs=[
            pl.BlockSpec((BN, 1), lambda i, j: (i, 0)),
            pl.BlockSpec((BN, 1), lambda i, j: (i, 0)),
            pl.BlockSpec((1, BM), lambda i, j: (0, j)),
            pl.BlockSpec((1, BM), lambda i, j: (0, j)),
        ],
        out_specs=[
            pl.BlockSpec((BN, 1), lambda i, j: (i, 0)),
            pl.BlockSpec((BN, 1), lambda i, j: (i, 0)),
            pl.BlockSpec((1, M), lambda i, j: (0, 0)),
        ],
        out_shape=[
            jax.ShapeDtypeStruct((NTC, 1), jnp.float32),
            jax.ShapeDtypeStruct((NTC, 1), jnp.float32),
            jax.ShapeDtypeStruct((1, M), jnp.float32),
        ],
        scratch_shapes=[
            pltpu.VMEM((BN, 1), jnp.float32),
            pltpu.VMEM((1, M), jnp.float32),
            pltpu.VMEM((BN, 1), jnp.float32),
            pltpu.VMEM((BN, 1), jnp.float32),
        ],
    )(bx, by, rx, ry)


def _finalize_body(rm2sc_ref, mksc_ref, rm2tc_ref, mktc_ref,
                   colsc_ref, coltc_ref, out_ref):
    colmin2 = jnp.minimum(jnp.min(colsc_ref[...], axis=0, keepdims=True),
                          coltc_ref[...])
    sum2 = jnp.sum(jnp.sqrt(jnp.maximum(colmin2, 0.0)))
    mksc = mksc_ref[...]
    rdsc = jnp.sqrt(jnp.maximum(rm2sc_ref[...], 0.0)) * mksc
    mktc = mktc_ref[...]
    rdtc = jnp.sqrt(jnp.maximum(rm2tc_ref[...], 0.0)) * mktc
    sum1 = jnp.sum(rdsc) + jnp.sum(rdtc)
    cnt = jnp.sum(mksc) + jnp.sum(mktc)
    mean1 = sum1 / jnp.maximum(cnt, 1.0)
    mean2 = sum2 / jnp.float32(M)
    out_ref[0, 0] = (mean1 + mean2) * 0.5


@jax.jit
def _centerline_loss(bez, ref):
    bx = bez[:, 0]
    by = bez[:, 1]
    rx = ref[:, 1]
    ry = ref[:, 0]

    rowmin2_sc, maskf_sc, colpart_sc = _make_sc_call()(
        bx[NTC:], by[NTC:], rx, ry)

    rowmin2_tc, maskf_tc, colpart_tc = _tc_call(
        bx[:NTC].reshape(NTC, 1), by[:NTC].reshape(NTC, 1),
        rx.reshape(1, M), ry.reshape(1, M))

    out = pl.pallas_call(
        _finalize_body,
        out_specs=pl.BlockSpec(memory_space=pltpu.SMEM),
        out_shape=jax.ShapeDtypeStruct((1, 1), jnp.float32),
    )(rowmin2_sc.reshape(NSC // 128, 128), maskf_sc.reshape(NSC // 128, 128),
      rowmin2_tc.reshape(NTC // 128, 128), maskf_tc.reshape(NTC // 128, 128),
      colpart_sc, colpart_tc)
    return out[0, 0]


def kernel(bezier_proj_centerline_img, ref_catheter_centerline):
    return _centerline_loss(bezier_proj_centerline_img,
                            ref_catheter_centerline)


# speedup vs baseline: 1.1145x; 1.1145x over previous
import functools

import jax
import jax.numpy as jnp
from jax import lax
from jax.experimental import pallas as pl
from jax.experimental.pallas import tpu as pltpu
from jax.experimental.pallas import tpu_sc as plsc

N = 8192
M = 8192
BIG = 3.0e38
FAR = 1.0e18

NTC = 5120
NSC = N - NTC

NW = 32
BPW = NSC // NW
L = 16
RV = M // L
UNROLL = 8
GB = 4

BN = 512
BM = 2048


_GATHER_DNUMS = lax.GatherDimensionNumbers(
    offset_dims=(), collapsed_slice_dims=(0,), start_index_map=(0,))


def _shuffle(v, idx):
    return lax.gather(v, idx[:, None], _GATHER_DNUMS, slice_sizes=(1,),
                      mode=lax.GatherScatterMode.PROMISE_IN_BOUNDS)


def _lane_min_all(v):
    lane_iota = lax.iota(jnp.int32, L)
    for s in (8, 4, 2, 1):
        v = jnp.minimum(v, _shuffle(v, lane_iota ^ s))
    return v


def _sc_main(bx_hbm, by_hbm, rx_hbm, ry_hbm,
             rowmin_hbm, maskf_hbm, colpart_hbm,
             rx_v, ry_v, rn2_v, colacc_v, bx_v, by_v, rmin_v, mask_v):
    wid = lax.axis_index("c") * 16 + lax.axis_index("s")
    base = wid * BPW

    pltpu.sync_copy(rx_hbm, rx_v)
    pltpu.sync_copy(ry_hbm, ry_v)
    pltpu.sync_copy(bx_hbm.at[pl.ds(base, BPW)], bx_v)
    pltpu.sync_copy(by_hbm.at[pl.ds(base, BPW)], by_v)

    def mask_body(t, carry):
        sl = pl.ds(t * L, L)
        bxv = bx_v[sl]
        byv = by_v[sl]
        ok = ((bxv >= -2000.0) & (bxv <= 2000.0) &
              (byv >= -2000.0) & (byv <= 2000.0))
        mask_v[sl] = jnp.where(ok, jnp.float32(1.0), jnp.float32(0.0))
        bx_v[sl] = jnp.where(ok, bxv, jnp.float32(FAR))
        by_v[sl] = jnp.where(ok, byv, jnp.float32(FAR))
        return carry

    lax.fori_loop(0, BPW // L, mask_body, 0)

    def prep_body(j, carry):
        sl = pl.ds(j * L, L)
        rxv = rx_v[sl]
        ryv = ry_v[sl]
        rn2_v[sl] = rxv * rxv + ryv * ryv
        colacc_v[sl] = jnp.full((L,), BIG, jnp.float32)
        return carry

    lax.fori_loop(0, RV, prep_body, 0)

    lane_iota = lax.iota(jnp.int32, L)

    def super_body(t, carry):
        i0 = t * L
        bxv16 = bx_v[pl.ds(i0, L)]
        byv16 = by_v[pl.ds(i0, L)]
        rminvec = jnp.full((L,), BIG, jnp.float32)
        for sub in range(L // GB):
            nbx = []
            nby = []
            bn2 = []
            for k in range(GB):
                lane = sub * GB + k
                bxb = jnp.full((L,), bxv16[lane], jnp.float32)
                byb = jnp.full((L,), byv16[lane], jnp.float32)
                nbx.append(bxb * (-2.0))
                nby.append(byb * (-2.0))
                bn2.append(bxb * bxb + byb * byb)

            def inner(j, accs):
                accs = list(accs)
                sl = pl.ds(j * L, L)
                rxv = rx_v[sl]
                ryv = ry_v[sl]
                rn2v = rn2_v[sl]
                d2s = []
                for k in range(GB):
                    t1 = nbx[k] * rxv + rn2v
                    t2 = nby[k] * ryv + t1
                    accs[k] = jnp.minimum(accs[k], t2)
                    d2s.append(t2 + bn2[k])
                cm = jnp.minimum(jnp.minimum(d2s[0], d2s[1]),
                                 jnp.minimum(d2s[2], d2s[3]))
                colacc_v[sl] = jnp.minimum(colacc_v[sl], cm)
                return tuple(accs)

            accs = plsc.parallel_loop(
                0, RV, 1, unroll=UNROLL,
                carry=tuple(jnp.full((L,), BIG, jnp.float32)
                            for _ in range(GB)))(inner)
            for k in range(GB):
                lane = sub * GB + k
                red = _lane_min_all(accs[k]) + bn2[k]
                rminvec = jnp.where(lane_iota == lane, red, rminvec)
        rmin_v[pl.ds(i0, L)] = rminvec
        return carry

    lax.fori_loop(0, BPW // L, super_body, 0)

    pltpu.sync_copy(rmin_v, rowmin_hbm.at[pl.ds(base, BPW)])
    pltpu.sync_copy(mask_v, maskf_hbm.at[pl.ds(base, BPW)])
    pltpu.sync_copy(colacc_v, colpart_hbm.at[wid])


def _make_sc_call():
    return functools.partial(
        pl.kernel,
        out_type=(jax.ShapeDtypeStruct((NSC,), jnp.float32),
                  jax.ShapeDtypeStruct((NSC,), jnp.float32),
                  jax.ShapeDtypeStruct((NW, M), jnp.float32)),
        mesh=plsc.VectorSubcoreMesh(core_axis_name="c", subcore_axis_name="s",
                                    num_cores=2, num_subcores=16),
        scratch_types=[
            pltpu.VMEM((M,), jnp.float32),
            pltpu.VMEM((M,), jnp.float32),
            pltpu.VMEM((M,), jnp.float32),
            pltpu.VMEM((M,), jnp.float32),
            pltpu.VMEM((BPW,), jnp.float32),
            pltpu.VMEM((BPW,), jnp.float32),
            pltpu.VMEM((BPW,), jnp.float32),
            pltpu.VMEM((BPW,), jnp.float32),
        ],
    )(_sc_main)


def _tc_body(bx_ref, by_ref, rx_ref, ry_ref,
             rowmin_ref, maskf_ref, colpart_ref, rowacc, colacc,
             bxm_s, bym_s):
    i = pl.program_id(0)
    j = pl.program_id(1)
    ni = pl.num_programs(0)
    nj = pl.num_programs(1)

    rx = rx_ref[...]
    ry = ry_ref[...]

    @pl.when(j == 0)
    def _():
        bx = bx_ref[...]
        by = by_ref[...]
        mask = ((bx >= -2000.0) & (bx <= 2000.0) &
                (by >= -2000.0) & (by <= 2000.0))
        bxm_s[...] = jnp.where(mask, bx, FAR)
        bym_s[...] = jnp.where(mask, by, FAR)
        maskf_ref[...] = mask.astype(jnp.float32)

    bxm = bxm_s[...]
    bym = bym_s[...]

    dx = bxm - rx
    dy = bym - ry
    d2 = dx * dx + dy * dy

    rmin = jnp.min(d2, axis=1, keepdims=True)

    @pl.when(j == 0)
    def _():
        rowacc[...] = rmin

    @pl.when(j != 0)
    def _():
        rowacc[...] = jnp.minimum(rowacc[...], rmin)

    cmin = jnp.min(d2, axis=0, keepdims=True)

    @pl.when(i == 0)
    def _():
        colacc[0:1, pl.ds(j * BM, BM)] = cmin

    @pl.when(i != 0)
    def _():
        colacc[0:1, pl.ds(j * BM, BM)] = jnp.minimum(
            colacc[0:1, pl.ds(j * BM, BM)], cmin)

    @pl.when(j == nj - 1)
    def _():
        rowmin_ref[...] = rowacc[...]

    @pl.when((i == ni - 1) & (j == nj - 1))
    def _():
        colpart_ref[...] = colacc[...]


def _tc_call(bx, by, rx, ry):
    grid = (NTC // BN, M // BM)
    return pl.pallas_call(
        _tc_body,
        grid=grid,
        in_specs=[
            pl.BlockSpec((BN, 1), lambda i, j: (i, 0)),
            pl.BlockSpec((BN, 1), lambda i, j: (i, 0)),
            pl.BlockSpec((1, BM), lambda i, j: (0, j)),
            pl.BlockSpec((1, BM), lambda i, j: (0, j)),
        ],
        out_specs=[
            pl.BlockSpec((BN, 1), lambda i, j: (i, 0)),
            pl.BlockSpec((BN, 1), lambda i, j: (i, 0)),
            pl.BlockSpec((1, M), lambda i, j: (0, 0)),
        ],
        out_shape=[
            jax.ShapeDtypeStruct((NTC, 1), jnp.float32),
            jax.ShapeDtypeStruct((NTC, 1), jnp.float32),
            jax.ShapeDtypeStruct((1, M), jnp.float32),
        ],
        scratch_shapes=[
            pltpu.VMEM((BN, 1), jnp.float32),
            pltpu.VMEM((1, M), jnp.float32),
            pltpu.VMEM((BN, 1), jnp.float32),
            pltpu.VMEM((BN, 1), jnp.float32),
        ],
    )(bx, by, rx, ry)


def _finalize_body(rm2sc_ref, mksc_ref, rm2tc_ref, mktc_ref,
                   colsc_ref, coltc_ref, out_ref):
    colmin2 = jnp.minimum(jnp.min(colsc_ref[...], axis=0, keepdims=True),
                          coltc_ref[...])
    sum2 = jnp.sum(jnp.sqrt(jnp.maximum(colmin2, 0.0)))
    mksc = mksc_ref[...]
    rdsc = jnp.sqrt(jnp.maximum(rm2sc_ref[...], 0.0)) * mksc
    mktc = mktc_ref[...]
    rdtc = jnp.sqrt(jnp.maximum(rm2tc_ref[...], 0.0)) * mktc
    sum1 = jnp.sum(rdsc) + jnp.sum(rdtc)
    cnt = jnp.sum(mksc) + jnp.sum(mktc)
    mean1 = sum1 / jnp.maximum(cnt, 1.0)
    mean2 = sum2 / jnp.float32(M)
    out_ref[0, 0] = (mean1 + mean2) * 0.5


@jax.jit
def _centerline_loss(bez, ref):
    bx = bez[:, 0]
    by = bez[:, 1]
    rx = ref[:, 1]
    ry = ref[:, 0]

    rowmin2_sc, maskf_sc, colpart_sc = _make_sc_call()(
        bx[NTC:], by[NTC:], rx, ry)

    rowmin2_tc, maskf_tc, colpart_tc = _tc_call(
        bx[:NTC].reshape(NTC, 1), by[:NTC].reshape(NTC, 1),
        rx.reshape(1, M), ry.reshape(1, M))

    out = pl.pallas_call(
        _finalize_body,
        out_specs=pl.BlockSpec(memory_space=pltpu.SMEM),
        out_shape=jax.ShapeDtypeStruct((1, 1), jnp.float32),
    )(rowmin2_sc.reshape(NSC // 128, 128), maskf_sc.reshape(NSC // 128, 128),
      rowmin2_tc.reshape(NTC // 128, 128), maskf_tc.reshape(NTC // 128, 128),
      colpart_sc, colpart_tc)
    return out[0, 0]


def kernel(bezier_proj_centerline_img, ref_catheter_centerline):
    return _centerline_loss(bezier_proj_centerline_img,
                            ref_catheter_centerline)
